# baseline (speedup 1.0000x reference)
import functools

import jax
import jax.numpy as jnp
from jax import lax
from jax.experimental import pallas as pl
from jax.experimental.pallas import tpu as pltpu
from jax.experimental.pallas import tpu_sc as plsc

EMB_DIM = 1024
LANES = 16
NUM_CORES = 2
NUM_SUBCORES = 16
NUM_WORKERS = NUM_CORES * NUM_SUBCORES


@jax.jit
def _lookup(segment, table):
    bsz, seq = segment.shape
    n = bsz * seq
    per_w = n // NUM_WORKERS
    w_per_row = seq // per_w
    n_groups = per_w // LANES
    mesh = plsc.VectorSubcoreMesh(core_axis_name="c", subcore_axis_name="s")

    @functools.partial(
        pl.kernel,
        out_type=jax.ShapeDtypeStruct((bsz, seq, EMB_DIM), jnp.float32),
        mesh=mesh,
        scratch_types=[
            pltpu.VMEM((1, per_w), jnp.int32),
            pltpu.VMEM((3, EMB_DIM), jnp.float32),
            pltpu.SemaphoreType.DMA,
        ],
    )
    def body(seg_hbm, table_hbm, out_hbm, idx_v, table_v, sem):
        wid = lax.axis_index("s") * NUM_CORES + lax.axis_index("c")
        row = wid // w_per_row
        col0 = (wid % w_per_row) * per_w
        pltpu.async_copy(table_hbm, table_v, sem)
        pltpu.async_copy(
            seg_hbm.at[pl.ds(row, 1), pl.ds(col0, per_w)], idx_v, sem
        )
        pltpu.make_async_copy(table_hbm, table_v, sem).wait()
        pltpu.make_async_copy(
            seg_hbm.at[pl.ds(row, 1), pl.ds(col0, per_w)], idx_v, sem
        ).wait()

        def group(g, carry):
            seg_vec = idx_v[0, pl.ds(g * LANES, LANES)]
            tok = col0 + g * LANES
            for r in range(LANES):
                pltpu.async_copy(
                    table_v.at[seg_vec[r]], out_hbm.at[row, tok + r], sem
                )
            return carry

        lax.fori_loop(0, n_groups, group, 0)

        pltpu.make_async_copy(
            out_hbm.at[pl.ds(row, 1), pl.ds(col0, per_w)],
            out_hbm.at[pl.ds(row, 1), pl.ds(col0, per_w)],
            sem,
        ).wait()

    return body(segment, table)


def kernel(segment, table):
    return _lookup(segment.astype(jnp.int32), table)


# speedup vs baseline: 1.0205x; 1.0205x over previous
import functools

import jax
import jax.numpy as jnp
from jax import lax
from jax.experimental import pallas as pl
from jax.experimental.pallas import tpu as pltpu
from jax.experimental.pallas import tpu_sc as plsc

EMB_DIM = 1024
LANES = 16
NUM_CORES = 2
NUM_SUBCORES = 16
NUM_WORKERS = NUM_CORES * NUM_SUBCORES
SPMEM_SHARE = 6


@jax.jit
def _lookup(segment, table):
    bsz, seq = segment.shape
    n = bsz * seq
    per_w = n // NUM_WORKERS
    w_per_row = seq // per_w
    n_groups = per_w // LANES
    mesh = plsc.VectorSubcoreMesh(core_axis_name="c", subcore_axis_name="s")

    @functools.partial(
        pl.kernel,
        out_type=jax.ShapeDtypeStruct((bsz, seq, EMB_DIM), jnp.float32),
        mesh=mesh,
        scratch_types=[
            pltpu.VMEM((1, per_w), jnp.int32),
            pltpu.VMEM((3, EMB_DIM), jnp.float32),
            pltpu.VMEM_SHARED((3, EMB_DIM), jnp.float32),
            pltpu.SemaphoreType.DMA,
        ],
    )
    def body(seg_hbm, table_hbm, out_hbm, idx_v, table_v, table_sh, sem):
        sid = lax.axis_index("s")
        wid = sid * NUM_CORES + lax.axis_index("c")
        row = wid // w_per_row
        col0 = (wid % w_per_row) * per_w
        pltpu.async_copy(table_hbm, table_v, sem)
        pltpu.async_copy(
            seg_hbm.at[pl.ds(row, 1), pl.ds(col0, per_w)], idx_v, sem
        )

        @pl.when(sid == 0)
        def _():
            pltpu.sync_copy(table_hbm, table_sh)

        pltpu.make_async_copy(table_hbm, table_v, sem).wait()
        pltpu.make_async_copy(
            seg_hbm.at[pl.ds(row, 1), pl.ds(col0, per_w)], idx_v, sem
        ).wait()
        plsc.subcore_barrier()

        def group(g, carry):
            seg_vec = idx_v[0, pl.ds(g * LANES, LANES)]
            tok = col0 + g * LANES
            for r in range(LANES):
                src = table_sh if r < SPMEM_SHARE else table_v
                pltpu.async_copy(
                    src.at[seg_vec[r]], out_hbm.at[row, tok + r], sem
                )
            return carry

        lax.fori_loop(0, n_groups, group, 0)

        pltpu.make_async_copy(
            out_hbm.at[pl.ds(row, 1), pl.ds(col0, per_w)],
            out_hbm.at[pl.ds(row, 1), pl.ds(col0, per_w)],
            sem,
        ).wait()

    return body(segment, table)


def kernel(segment, table):
    return _lookup(segment.astype(jnp.int32), table)
